# baseline (speedup 1.0000x reference)
import functools

import numpy as np
import jax
import jax.numpy as jnp
from jax import lax
from jax.experimental import pallas as pl
from jax.experimental.pallas import tpu as pltpu
from jax.experimental.pallas import tpu_sc as plsc

_VOCAB = 1000000
_D = 64
_B = 4096
_S = 200
_MAX_LEN = 5000

_NC = 2
_NS = 16
_NW = _NC * _NS
_SEQ_PER_W = _B // _NW


def _pos_encoding() -> np.ndarray:
    pos = np.arange(_MAX_LEN, dtype=np.float32)[:, None]
    i = np.arange(_D, dtype=np.float32)[None, :]
    angle_rates = 1.0 / np.power(
        10000.0, (2.0 * np.floor(i / 2.0)) / np.float32(_D))
    angle_rads = pos * angle_rates
    pe = np.zeros((_MAX_LEN, _D), dtype=np.float32)
    pe[:, 0::2] = np.sin(angle_rads[:, 0::2])
    pe[:, 1::2] = np.cos(angle_rads[:, 1::2])
    return pe[:_S]


_PE = _pos_encoding()


def _emb_kernel(idx_hbm, table_hbm, pe_hbm, out_hbm, idx_v, rows_v, pe_v, sem):
    wid = lax.axis_index("s") * _NC + lax.axis_index("c")
    pltpu.sync_copy(pe_hbm, pe_v)

    def seq_body(q, carry):
        seq = wid * _SEQ_PER_W + q
        pltpu.sync_copy(idx_hbm.at[pl.ds(seq * 2, 2)], idx_v)
        cp0 = pltpu.async_copy(
            table_hbm.at[idx_v.at[0]], rows_v.at[pl.ds(0, 100)], sem)
        cp1 = pltpu.async_copy(
            table_hbm.at[idx_v.at[1]], rows_v.at[pl.ds(100, 100)], sem)
        cp0.wait()
        cp1.wait()

        def add_body(r, c):
            for d in range(_D // 16):
                sl = pl.ds(d * 16, 16)
                rows_v[r, sl] = rows_v[r, sl] + pe_v[r, sl]
            return c

        lax.fori_loop(0, _S, add_body, 0, unroll=4)
        pltpu.sync_copy(rows_v, out_hbm.at[pl.ds(seq * _S, _S)])
        return carry

    lax.fori_loop(0, _SEQ_PER_W, seq_body, 0)


@jax.jit
def _run(idx, table, pe):
    mesh = plsc.VectorSubcoreMesh(core_axis_name="c", subcore_axis_name="s")
    f = functools.partial(
        pl.kernel,
        mesh=mesh,
        out_type=jax.ShapeDtypeStruct((_B * _S, _D), jnp.float32),
        scratch_types=[
            pltpu.VMEM((2, 100), jnp.int32),
            pltpu.VMEM((_S, _D), jnp.float32),
            pltpu.VMEM((_S, _D), jnp.float32),
            pltpu.SemaphoreType.DMA,
        ],
        compiler_params=pltpu.CompilerParams(use_tc_tiling_on_sc=False),
    )(_emb_kernel)
    return f(idx, table, pe)


def kernel(x, table):
    idx = x.astype(jnp.int32).reshape(_B * 2, 100)
    out = _run(idx, table, jnp.asarray(_PE))
    return out.reshape(_B, _S, _D)
